# baseline (speedup 1.0000x reference)
import dataclasses
import functools

import jax
import jax.numpy as jnp
from jax import lax
from jax.experimental import pallas as pl
from jax.experimental.pallas import tpu as pltpu
from jax.experimental.pallas import tpu_sc as plsc

N_USERS = 25000
N_ITEMS = 25000
N_TOTAL = 50000
EMB = 64
FEAT_COLS = N_TOTAL + 2
N_LAYERS = 3
E = 800000
BATCH = 4096

NC = 2
NS = 16
BLK = 128
NB = 400
E_PAD = NC * 0 + NS * NB * BLK
HALF = 25000
HPAD = 25008
NP = 2 * HPAD
STRIPE = HPAD // NS
DEG_ROWS = 3200
DEG_CHUNK = 3200
SENT = N_TOTAL + 16

_mesh = plsc.VectorSubcoreMesh(core_axis_name="c", subcore_axis_name="s")

_cp = pltpu.CompilerParams(needs_layout_passes=False,
                           use_tc_tiling_on_sc=False)


def _f32(shape):
    return jax.ShapeDtypeStruct(shape, jnp.float32)


@jax.jit
def _degrees(rows_feat, rows_adj, ident, zeros_deg):
    @functools.partial(
        pl.kernel,
        out_type=(_f32((DEG_ROWS, 16)), _f32((DEG_ROWS, 16))),
        mesh=_mesh,
        compiler_params=_cp,
        scratch_types=[
            pltpu.VMEM((DEG_ROWS, 16), jnp.float32),
            pltpu.VMEM((DEG_ROWS, 16), jnp.float32),
            pltpu.VMEM((DEG_CHUNK,), jnp.int32),
            pltpu.VMEM((25, BLK), jnp.int32),
            pltpu.VMEM_SHARED((DEG_ROWS, 16), jnp.float32),
            pltpu.VMEM_SHARED((DEG_ROWS, 16), jnp.float32),
        ],
    )
    def k(rf_hbm, ra_hbm, id_hbm, z_hbm, outf_hbm, outa_hbm,
          locf, loca, idxv, identv, accf, acca):
        c = lax.axis_index("c")
        s = lax.axis_index("s")
        ones = jnp.full((16,), 1.0, jnp.float32)

        pltpu.sync_copy(z_hbm, locf)
        pltpu.sync_copy(z_hbm, loca)
        pltpu.sync_copy(id_hbm, identv)
        pltpu.sync_copy(z_hbm.at[pl.ds(s * (DEG_ROWS // NS), DEG_ROWS // NS)],
                        accf.at[pl.ds(s * (DEG_ROWS // NS), DEG_ROWS // NS)])
        pltpu.sync_copy(z_hbm.at[pl.ds(s * (DEG_ROWS // NS), DEG_ROWS // NS)],
                        acca.at[pl.ds(s * (DEG_ROWS // NS), DEG_ROWS // NS)])

        edges_per_sub = E_PAD // NS
        n_chunks = edges_per_sub // DEG_CHUNK

        def scan_table(rows_hbm, loc):
            @pl.loop(0, n_chunks)
            def _(t):
                base = s * edges_per_sub + t * DEG_CHUNK
                pltpu.sync_copy(rows_hbm.at[pl.ds(base, DEG_CHUNK)], idxv)

                @pl.loop(0, DEG_CHUNK // 16)
                def _(u):
                    v = idxv[pl.ds(u * 16, 16)]
                    hi = lax.shift_right_logical(v, 4)
                    lo = lax.bitwise_and(v, jnp.full((16,), 15, jnp.int32))
                    plsc.addupdate_scatter(loc, [hi, lo], ones)

        scan_table(rf_hbm, locf)
        scan_table(ra_hbm, loca)

        plsc.subcore_barrier()
        for kk in range(DEG_ROWS // BLK):
            pltpu.sync_copy(locf.at[pl.ds(kk * BLK, BLK)],
                            accf.at[identv.at[kk]], add=True)
            pltpu.sync_copy(loca.at[pl.ds(kk * BLK, BLK)],
                            acca.at[identv.at[kk]], add=True)
        plsc.subcore_barrier()

        rpw = DEG_ROWS // NS
        pltpu.sync_copy(accf.at[pl.ds(s * rpw, rpw)],
                        outf_hbm.at[pl.ds(s * rpw, rpw)])
        pltpu.sync_copy(acca.at[pl.ds(s * rpw, rpw)],
                        outa_hbm.at[pl.ds(s * rpw, rpw)])

    return k(rows_feat, rows_adj, ident, zeros_deg)


KB = 10
NCHUNK = NB // KB
NBLK_TOT = E_PAD // BLK


def _make_agg(vt):
    @jax.jit
    def agg(table, cols, ldst, zeros_half):
        @functools.partial(
            pl.kernel,
            out_type=_f32((NP, EMB)),
            mesh=_mesh,
            compiler_params=_cp,
            scratch_types=[
                pltpu.VMEM((KB, BLK), jnp.int32),
                pltpu.VMEM((KB, BLK), jnp.int32),
                pltpu.VMEM((KB, BLK), jnp.int32),
                pltpu.VMEM((KB, BLK), jnp.int32),
                pltpu.VMEM((BLK, EMB), jnp.float32),
                pltpu.VMEM((BLK, EMB), jnp.float32),
                pltpu.VMEM((BLK, EMB), jnp.float32),
                pltpu.VMEM_SHARED((HPAD, EMB), jnp.float32),
                pltpu.SemaphoreType.DMA,
                pltpu.SemaphoreType.DMA,
                pltpu.SemaphoreType.DMA,
                pltpu.SemaphoreType.DMA,
                pltpu.SemaphoreType.DMA,
            ],
        )
        def k(tab_hbm, cols_hbm, ldst_hbm, z_hbm, out_hbm,
              colv0, colv1, ldstv0, ldstv1, rbuf0, rbuf1, rbuf2, acc,
              si0, si1, sg0, sg1, sg2):
            c = lax.axis_index("c")
            s = lax.axis_index("s")
            colv = (colv0, colv1)
            ldstv = (ldstv0, ldstv1)
            rbuf = (rbuf0, rbuf1, rbuf2)
            semi = (si0, si1)
            semg = (sg0, sg1, sg2)

            pltpu.sync_copy(z_hbm.at[pl.ds(s * STRIPE, STRIPE)],
                            acc.at[pl.ds(s * STRIPE, STRIPE)])

            def idx_args(k_, slot):
                blk0 = s * NB + k_ * KB
                return ((cols_hbm.at[pl.ds(blk0, KB)],
                         colv[slot], semi[slot]),
                        (ldst_hbm.at[pl.ds(c * NBLK_TOT + blk0, KB)],
                         ldstv[slot], semi[slot]))

            def idx_load(k_, slot):
                a, b = idx_args(k_, slot)
                pltpu.async_copy(*a)
                pltpu.async_copy(*b)

            def idx_wait(k_, slot):
                a, b = idx_args(k_, slot)
                pltpu.make_async_copy(*a).wait()
                pltpu.make_async_copy(*b).wait()

            def g_args(k_, slot, kb, rs):
                return (tab_hbm.at[colv[slot].at[kb]],
                        rbuf[rs], semg[rs])

            idx_load(0, 0)
            idx_load(1, 1)
            plsc.subcore_barrier()

            @pl.loop(0, NCHUNK // 2)
            def _(kp):
                for par in (0, 1):
                    k_ = kp * 2 + par
                    idx_wait(k_, par)
                    pltpu.async_copy(*g_args(k_, par, 0, 0))
                    pltpu.async_copy(*g_args(k_, par, 1, 1))
                    for kb in range(KB):
                        rs = kb % 3
                        if kb + 2 < KB:
                            pltpu.async_copy(
                                *g_args(k_, par, kb + 2, (kb + 2) % 3))
                        pltpu.make_async_copy(*g_args(k_, par, kb, rs)).wait()
                        pltpu.sync_copy(rbuf[rs],
                                        acc.at[ldstv[par].at[kb]],
                                        add=True)

                    @pl.when(k_ + 2 < NCHUNK)
                    def _():
                        idx_load(k_ + 2, par)

            plsc.subcore_barrier()
            pltpu.sync_copy(
                acc.at[pl.ds(s * STRIPE, STRIPE)],
                out_hbm.at[pl.ds(c * HPAD + s * STRIPE, STRIPE)])

        return k(table, cols, ldst, zeros_half)

    return agg


_agg_feat = _make_agg(FEAT_COLS)
_agg_adj = _make_agg(NP)


@jax.jit
def _batch_gather(table, bidx):
    NW = NC * NS

    @functools.partial(
        pl.kernel,
        out_type=(_f32((BATCH, EMB)), _f32((BATCH, EMB)), _f32((BATCH, EMB))),
        mesh=_mesh,
        compiler_params=_cp,
        scratch_types=[
            pltpu.VMEM((3, BLK), jnp.int32),
            pltpu.VMEM((BLK, EMB), jnp.float32),
            pltpu.SemaphoreType.DMA,
        ],
    )
    def k(tab_hbm, idx_hbm, o0_hbm, o1_hbm, o2_hbm, idxv, rbuf, sem):
        c = lax.axis_index("c")
        s = lax.axis_index("s")
        wid = s * NC + c
        outs = (o0_hbm, o1_hbm, o2_hbm)
        for j in range(3):
            pltpu.sync_copy(idx_hbm.at[pl.ds(wid + 32 * j, 1)],
                            idxv.at[pl.ds(j, 1)])
        for j in range(3):
            pltpu.async_copy(tab_hbm.at[idxv.at[j]], rbuf, sem).wait()
            pltpu.sync_copy(rbuf, outs[j].at[pl.ds(wid * BLK, BLK)])

    return k(table, bidx)


@jax.jit
def _l2_norms(u, p, n):
    def body(u_ref, p_ref, n_ref, o_ref):
        uu = u_ref[...]
        pp = p_ref[...]
        nn = n_ref[...]
        o_ref[...] = jnp.sum(uu * uu + pp * pp + nn * nn, axis=1,
                             keepdims=True)

    out = pl.pallas_call(body, out_shape=_f32((BATCH, 1)))(u, p, n)
    return out[:, 0]


def _pad_half_layout(v):
    return jnp.pad(v.reshape(2, HALF), ((0, 0), (0, HPAD - HALF))).reshape(-1)


def _ldst_both(rows):
    l0 = jnp.where(rows < HALF, rows, HALF)
    r1 = rows - HALF
    l1 = jnp.where((r1 >= 0) & (r1 < HALF), r1, HALF)
    return jnp.stack([l0, l1]).reshape(2, E_PAD // BLK, BLK)


@jax.jit
def kernel(embedding_weight, feat_row, feat_col, adj_row, adj_col,
           users, pos_items, neg_items):
    i32 = jnp.int32
    pad_n = E_PAD - E
    rows_f = jnp.concatenate([feat_row, jnp.full((pad_n,), SENT, i32)])
    rows_a = jnp.concatenate([adj_row, jnp.full((pad_n,), SENT, i32)])
    cols_f = jnp.concatenate(
        [feat_col, jnp.zeros((pad_n,), i32)]).reshape(NBLK_TOT, BLK)
    adj_col_p = adj_col + 8 * (adj_col >= HALF).astype(i32)
    cols_a = jnp.concatenate(
        [adj_col_p, jnp.zeros((pad_n,), i32)]).reshape(NBLK_TOT, BLK)
    ldst_f = _ldst_both(rows_f)
    ldst_a = _ldst_both(rows_a)
    ldst_f = ldst_f.reshape(2 * NBLK_TOT, BLK)
    ldst_a = ldst_a.reshape(2 * NBLK_TOT, BLK)

    ident = jnp.arange(DEG_ROWS, dtype=i32).reshape(DEG_ROWS // BLK, BLK)
    zeros_deg = jnp.zeros((DEG_ROWS, 16), jnp.float32)
    zeros_half = jnp.zeros((HPAD, EMB), jnp.float32)

    degf2, dega2 = _degrees(rows_f, rows_a, ident, zeros_deg)
    deg_f = degf2.reshape(-1)[:N_TOTAL]
    deg_a = dega2.reshape(-1)[:N_TOTAL]

    x_raw = _agg_feat(embedding_weight, cols_f, ldst_f, zeros_half)
    f_feat = _pad_half_layout(jnp.where(deg_f > 0, 1.0 / deg_f, 0.0))
    x = x_raw * f_feat[:, None]

    g_a = _pad_half_layout(
        jnp.where(deg_a > 0, 1.0 / jnp.sqrt(deg_a), 0.0))

    rep_sum = x
    h = x
    for _ in range(N_LAYERS):
        s_raw = _agg_adj(h * g_a[:, None], cols_a, ldst_a, zeros_half)
        h = s_raw * g_a[:, None]
        rep_sum = rep_sum + h
    final_rep = rep_sum * 0.25

    upos = users
    ppos = HPAD + pos_items
    npos = HPAD + neg_items
    bidx = jnp.concatenate([upos, ppos, npos]).reshape(
        (3 * BATCH) // BLK, BLK)
    users_r, pos_items_r, neg_items_r = _batch_gather(final_rep, bidx)
    l2_norm_sq = _l2_norms(users_r, pos_items_r, neg_items_r)
    return users_r, pos_items_r, neg_items_r, l2_norm_sq


# speedup vs baseline: 1.1795x; 1.1795x over previous
import dataclasses
import functools

import jax
import jax.numpy as jnp
from jax import lax
from jax.experimental import pallas as pl
from jax.experimental.pallas import tpu as pltpu
from jax.experimental.pallas import tpu_sc as plsc

N_USERS = 25000
N_ITEMS = 25000
N_TOTAL = 50000
EMB = 64
FEAT_COLS = N_TOTAL + 2
N_LAYERS = 3
E = 800000
BATCH = 4096

NC = 2
NS = 16
BLK = 128
NB = 400
E_PAD = NC * 0 + NS * NB * BLK
HALF = 25000
HPAD = 25008
NP = 2 * HPAD
STRIPE = HPAD // NS
DEG_ROWS = 3200
DEG_CHUNK = 3200
SENT = N_TOTAL + 16

_mesh = plsc.VectorSubcoreMesh(core_axis_name="c", subcore_axis_name="s")

_cp = pltpu.CompilerParams(needs_layout_passes=False,
                           use_tc_tiling_on_sc=False)


def _f32(shape):
    return jax.ShapeDtypeStruct(shape, jnp.float32)


@jax.jit
def _degrees(rows_feat, rows_adj, ident, zeros_deg):
    @functools.partial(
        pl.kernel,
        out_type=(_f32((DEG_ROWS, 16)), _f32((DEG_ROWS, 16))),
        mesh=_mesh,
        compiler_params=_cp,
        scratch_types=[
            pltpu.VMEM((DEG_ROWS, 16), jnp.float32),
            pltpu.VMEM((DEG_ROWS, 16), jnp.float32),
            pltpu.VMEM((DEG_CHUNK,), jnp.int32),
            pltpu.VMEM((25, BLK), jnp.int32),
            pltpu.VMEM_SHARED((DEG_ROWS, 16), jnp.float32),
            pltpu.VMEM_SHARED((DEG_ROWS, 16), jnp.float32),
        ],
    )
    def k(rf_hbm, ra_hbm, id_hbm, z_hbm, outf_hbm, outa_hbm,
          locf, loca, idxv, identv, accf, acca):
        c = lax.axis_index("c")
        s = lax.axis_index("s")
        ones = jnp.full((16,), 1.0, jnp.float32)

        pltpu.sync_copy(z_hbm, locf)
        pltpu.sync_copy(z_hbm, loca)
        pltpu.sync_copy(id_hbm, identv)
        pltpu.sync_copy(z_hbm.at[pl.ds(s * (DEG_ROWS // NS), DEG_ROWS // NS)],
                        accf.at[pl.ds(s * (DEG_ROWS // NS), DEG_ROWS // NS)])
        pltpu.sync_copy(z_hbm.at[pl.ds(s * (DEG_ROWS // NS), DEG_ROWS // NS)],
                        acca.at[pl.ds(s * (DEG_ROWS // NS), DEG_ROWS // NS)])

        edges_per_sub = E_PAD // NS
        n_chunks = edges_per_sub // DEG_CHUNK

        def scan_table(rows_hbm, loc):
            @pl.loop(0, n_chunks)
            def _(t):
                base = s * edges_per_sub + t * DEG_CHUNK
                pltpu.sync_copy(rows_hbm.at[pl.ds(base, DEG_CHUNK)], idxv)

                @pl.loop(0, DEG_CHUNK // 16)
                def _(u):
                    v = idxv[pl.ds(u * 16, 16)]
                    hi = lax.shift_right_logical(v, 4)
                    lo = lax.bitwise_and(v, jnp.full((16,), 15, jnp.int32))
                    plsc.addupdate_scatter(loc, [hi, lo], ones)

        scan_table(rf_hbm, locf)
        scan_table(ra_hbm, loca)

        plsc.subcore_barrier()
        for kk in range(DEG_ROWS // BLK):
            pltpu.sync_copy(locf.at[pl.ds(kk * BLK, BLK)],
                            accf.at[identv.at[kk]], add=True)
            pltpu.sync_copy(loca.at[pl.ds(kk * BLK, BLK)],
                            acca.at[identv.at[kk]], add=True)
        plsc.subcore_barrier()

        rpw = DEG_ROWS // NS
        pltpu.sync_copy(accf.at[pl.ds(s * rpw, rpw)],
                        outf_hbm.at[pl.ds(s * rpw, rpw)])
        pltpu.sync_copy(acca.at[pl.ds(s * rpw, rpw)],
                        outa_hbm.at[pl.ds(s * rpw, rpw)])

    return k(rows_feat, rows_adj, ident, zeros_deg)


KB = 10
NCHUNK = NB // KB
NBLK_TOT = E_PAD // BLK


def _make_agg(vt):
    @jax.jit
    def agg(table, cols, ldst, zeros_half, params):
        @functools.partial(
            pl.kernel,
            out_type=_f32((NP, EMB)),
            mesh=_mesh,
            compiler_params=_cp,
            scratch_types=[
                pltpu.VMEM((KB, BLK), jnp.int32),
                pltpu.VMEM((KB, BLK), jnp.int32),
                pltpu.VMEM((KB, BLK), jnp.int32),
                pltpu.VMEM((KB, BLK), jnp.int32),
                pltpu.VMEM((16,), jnp.int32),
                pltpu.VMEM((BLK, EMB), jnp.float32),
                pltpu.VMEM((BLK, EMB), jnp.float32),
                pltpu.VMEM((BLK, EMB), jnp.float32),
                pltpu.VMEM_SHARED((HPAD, EMB), jnp.float32),
                pltpu.SemaphoreType.DMA,
                pltpu.SemaphoreType.DMA,
                pltpu.SemaphoreType.DMA,
                pltpu.SemaphoreType.DMA,
                pltpu.SemaphoreType.DMA,
            ],
        )
        def k(tab_hbm, cols_hbm, ldst_hbm, z_hbm, par_hbm, out_hbm,
              colv0, colv1, ldstv0, ldstv1, pv, rbuf0, rbuf1, rbuf2, acc,
              si0, si1, sg0, sg1, sg2):
            c = lax.axis_index("c")
            s = lax.axis_index("s")
            colv = (colv0, colv1)
            ldstv = (ldstv0, ldstv1)
            rbuf = (rbuf0, rbuf1, rbuf2)
            semi = (si0, si1)
            semg = (sg0, sg1, sg2)

            pltpu.sync_copy(z_hbm.at[pl.ds(s * STRIPE, STRIPE)],
                            acc.at[pl.ds(s * STRIPE, STRIPE)])
            pltpu.sync_copy(par_hbm, pv)
            pvv = pv[...]
            start = jnp.where(c == 0, pvv[0], pvv[2])
            nch = jnp.where(c == 0, pvv[1], pvv[3])
            cnt = jnp.maximum(0, (nch - s + NS - 1) // NS)

            def idx_args(j, slot):
                blk0 = (start + s + NS * j) * KB
                return ((cols_hbm.at[pl.ds(blk0, KB)],
                         colv[slot], semi[slot]),
                        (ldst_hbm.at[pl.ds(c * NBLK_TOT + blk0, KB)],
                         ldstv[slot], semi[slot]))

            def idx_load(k_, slot):
                a, b = idx_args(k_, slot)
                pltpu.async_copy(*a)
                pltpu.async_copy(*b)

            def idx_wait(k_, slot):
                a, b = idx_args(k_, slot)
                pltpu.make_async_copy(*a).wait()
                pltpu.make_async_copy(*b).wait()

            def g_args(k_, slot, kb, rs):
                return (tab_hbm.at[colv[slot].at[kb]],
                        rbuf[rs], semg[rs])

            @pl.when(cnt > 0)
            def _():
                idx_load(0, 0)

            @pl.when(cnt > 1)
            def _():
                idx_load(1, 1)

            plsc.subcore_barrier()

            @pl.loop(0, (cnt + 1) // 2)
            def _(kp):
                for par in (0, 1):
                    j = kp * 2 + par

                    @pl.when(j < cnt)
                    def _():
                        idx_wait(j, par)
                        pltpu.async_copy(*g_args(j, par, 0, 0))
                        pltpu.async_copy(*g_args(j, par, 1, 1))
                        for kb in range(KB):
                            rs = kb % 3
                            if kb + 2 < KB:
                                pltpu.async_copy(
                                    *g_args(j, par, kb + 2, (kb + 2) % 3))
                            pltpu.make_async_copy(
                                *g_args(j, par, kb, rs)).wait()
                            pltpu.sync_copy(rbuf[rs],
                                            acc.at[ldstv[par].at[kb]],
                                            add=True)

                        @pl.when(j + 2 < cnt)
                        def _():
                            idx_load(j + 2, par)

            plsc.subcore_barrier()
            pltpu.sync_copy(
                acc.at[pl.ds(s * STRIPE, STRIPE)],
                out_hbm.at[pl.ds(c * HPAD + s * STRIPE, STRIPE)])

        return k(table, cols, ldst, zeros_half, params)

    return agg


_agg_feat = _make_agg(FEAT_COLS)
_agg_adj = _make_agg(NP)


@jax.jit
def _batch_gather(table, bidx):
    NW = NC * NS

    @functools.partial(
        pl.kernel,
        out_type=(_f32((BATCH, EMB)), _f32((BATCH, EMB)), _f32((BATCH, EMB))),
        mesh=_mesh,
        compiler_params=_cp,
        scratch_types=[
            pltpu.VMEM((3, BLK), jnp.int32),
            pltpu.VMEM((BLK, EMB), jnp.float32),
            pltpu.SemaphoreType.DMA,
        ],
    )
    def k(tab_hbm, idx_hbm, o0_hbm, o1_hbm, o2_hbm, idxv, rbuf, sem):
        c = lax.axis_index("c")
        s = lax.axis_index("s")
        wid = s * NC + c
        outs = (o0_hbm, o1_hbm, o2_hbm)
        for j in range(3):
            pltpu.sync_copy(idx_hbm.at[pl.ds(wid + 32 * j, 1)],
                            idxv.at[pl.ds(j, 1)])
        for j in range(3):
            pltpu.async_copy(tab_hbm.at[idxv.at[j]], rbuf, sem).wait()
            pltpu.sync_copy(rbuf, outs[j].at[pl.ds(wid * BLK, BLK)])

    return k(table, bidx)


@jax.jit
def _l2_norms(u, p, n):
    def body(u_ref, p_ref, n_ref, o_ref):
        uu = u_ref[...]
        pp = p_ref[...]
        nn = n_ref[...]
        o_ref[...] = jnp.sum(uu * uu + pp * pp + nn * nn, axis=1,
                             keepdims=True)

    out = pl.pallas_call(body, out_shape=_f32((BATCH, 1)))(u, p, n)
    return out[:, 0]


def _pad_half_layout(v):
    return jnp.pad(v.reshape(2, HALF), ((0, 0), (0, HPAD - HALF))).reshape(-1)


def _ldst_both(rows):
    l0 = jnp.where(rows < HALF, rows, HALF)
    r1 = rows - HALF
    l1 = jnp.where((r1 >= 0) & (r1 < HALF), r1, HALF)
    return jnp.stack([l0, l1]).reshape(2, E_PAD // BLK, BLK)


@jax.jit
def kernel(embedding_weight, feat_row, feat_col, adj_row, adj_col,
           users, pos_items, neg_items):
    i32 = jnp.int32
    pad_n = E_PAD - E
    rows_f = jnp.concatenate([feat_row, jnp.full((pad_n,), SENT, i32)])
    rows_a = jnp.concatenate([adj_row, jnp.full((pad_n,), SENT, i32)])
    cols_f = jnp.concatenate(
        [feat_col, jnp.zeros((pad_n,), i32)]).reshape(NBLK_TOT, BLK)
    ldst_f = _ldst_both(rows_f).reshape(2 * NBLK_TOT, BLK)

    adj_col_p = adj_col + 8 * (adj_col >= HALF).astype(i32)
    key = (adj_row >= HALF).astype(i32)
    n0 = jnp.int32(E) - jnp.sum(key, dtype=i32)
    perm = jnp.argsort(key, stable=True)
    rows_as = jnp.concatenate([adj_row[perm], jnp.full((pad_n,), SENT, i32)])
    cols_a = jnp.concatenate(
        [adj_col_p[perm], jnp.zeros((pad_n,), i32)]).reshape(NBLK_TOT, BLK)
    ldst_a = _ldst_both(rows_as).reshape(2 * NBLK_TOT, BLK)

    EPC = KB * BLK
    NCH_TOT = NBLK_TOT // KB
    c0n = (n0 + EPC - 1) // EPC
    c1s = n0 // EPC
    params_adj = (jnp.zeros((16,), i32).at[1].set(c0n)
                  .at[2].set(c1s).at[3].set(NCH_TOT - c1s))
    params_feat = jnp.zeros((16,), i32).at[1].set(NCH_TOT).at[3].set(NCH_TOT)

    ident = jnp.arange(DEG_ROWS, dtype=i32).reshape(DEG_ROWS // BLK, BLK)
    zeros_deg = jnp.zeros((DEG_ROWS, 16), jnp.float32)
    zeros_half = jnp.zeros((HPAD, EMB), jnp.float32)

    degf2, dega2 = _degrees(rows_f, rows_a, ident, zeros_deg)
    deg_f = degf2.reshape(-1)[:N_TOTAL]
    deg_a = dega2.reshape(-1)[:N_TOTAL]

    x_raw = _agg_feat(embedding_weight, cols_f, ldst_f, zeros_half,
                      params_feat)
    f_feat = _pad_half_layout(jnp.where(deg_f > 0, 1.0 / deg_f, 0.0))
    x = x_raw * f_feat[:, None]

    g_a = _pad_half_layout(
        jnp.where(deg_a > 0, 1.0 / jnp.sqrt(deg_a), 0.0))

    rep_sum = x
    h = x
    for _ in range(N_LAYERS):
        s_raw = _agg_adj(h * g_a[:, None], cols_a, ldst_a, zeros_half,
                         params_adj)
        h = s_raw * g_a[:, None]
        rep_sum = rep_sum + h
    final_rep = rep_sum * 0.25

    upos = users
    ppos = HPAD + pos_items
    npos = HPAD + neg_items
    bidx = jnp.concatenate([upos, ppos, npos]).reshape(
        (3 * BATCH) // BLK, BLK)
    users_r, pos_items_r, neg_items_r = _batch_gather(final_rep, bidx)
    l2_norm_sq = _l2_norms(users_r, pos_items_r, neg_items_r)
    return users_r, pos_items_r, neg_items_r, l2_norm_sq


# speedup vs baseline: 1.4690x; 1.2455x over previous
import dataclasses
import functools

import jax
import jax.numpy as jnp
from jax import lax
from jax.experimental import pallas as pl
from jax.experimental.pallas import tpu as pltpu
from jax.experimental.pallas import tpu_sc as plsc

N_USERS = 25000
N_ITEMS = 25000
N_TOTAL = 50000
EMB = 64
FEAT_COLS = N_TOTAL + 2
N_LAYERS = 3
E = 800000
BATCH = 4096

NC = 2
NS = 16
BLK = 128
NB = 400
E_PAD = NC * 0 + NS * NB * BLK
HALF = 25000
HPAD = 25008
NP = 2 * HPAD
STRIPE = HPAD // NS
DEG_ROWS = 3200
DEG_CHUNK = 3200
SENT = N_TOTAL + 16

_mesh = plsc.VectorSubcoreMesh(core_axis_name="c", subcore_axis_name="s")

_cp = pltpu.CompilerParams(needs_layout_passes=False,
                           use_tc_tiling_on_sc=False)


def _f32(shape):
    return jax.ShapeDtypeStruct(shape, jnp.float32)


@jax.jit
def _degrees(rows_feat, rows_adj, ident, zeros_deg):
    @functools.partial(
        pl.kernel,
        out_type=(_f32((DEG_ROWS, 16)), _f32((DEG_ROWS, 16))),
        mesh=_mesh,
        compiler_params=_cp,
        scratch_types=[
            pltpu.VMEM((DEG_ROWS, 16), jnp.float32),
            pltpu.VMEM((DEG_ROWS, 16), jnp.float32),
            pltpu.VMEM((DEG_CHUNK,), jnp.int32),
            pltpu.VMEM((25, BLK), jnp.int32),
            pltpu.VMEM_SHARED((DEG_ROWS, 16), jnp.float32),
            pltpu.VMEM_SHARED((DEG_ROWS, 16), jnp.float32),
        ],
    )
    def k(rf_hbm, ra_hbm, id_hbm, z_hbm, outf_hbm, outa_hbm,
          locf, loca, idxv, identv, accf, acca):
        c = lax.axis_index("c")
        s = lax.axis_index("s")
        ones = jnp.full((16,), 1.0, jnp.float32)

        pltpu.sync_copy(z_hbm, locf)
        pltpu.sync_copy(z_hbm, loca)
        pltpu.sync_copy(id_hbm, identv)
        pltpu.sync_copy(z_hbm.at[pl.ds(s * (DEG_ROWS // NS), DEG_ROWS // NS)],
                        accf.at[pl.ds(s * (DEG_ROWS // NS), DEG_ROWS // NS)])
        pltpu.sync_copy(z_hbm.at[pl.ds(s * (DEG_ROWS // NS), DEG_ROWS // NS)],
                        acca.at[pl.ds(s * (DEG_ROWS // NS), DEG_ROWS // NS)])

        edges_per_sub = E_PAD // NS
        n_chunks = edges_per_sub // DEG_CHUNK

        def scan_table(rows_hbm, loc):
            @pl.loop(0, n_chunks)
            def _(t):
                base = s * edges_per_sub + t * DEG_CHUNK
                pltpu.sync_copy(rows_hbm.at[pl.ds(base, DEG_CHUNK)], idxv)

                @pl.loop(0, DEG_CHUNK // 16)
                def _(u):
                    v = idxv[pl.ds(u * 16, 16)]
                    hi = lax.shift_right_logical(v, 4)
                    lo = lax.bitwise_and(v, jnp.full((16,), 15, jnp.int32))
                    plsc.addupdate_scatter(loc, [hi, lo], ones)

        scan_table(rf_hbm, locf)
        scan_table(ra_hbm, loca)

        plsc.subcore_barrier()
        for kk in range(DEG_ROWS // BLK):
            pltpu.sync_copy(locf.at[pl.ds(kk * BLK, BLK)],
                            accf.at[identv.at[kk]], add=True)
            pltpu.sync_copy(loca.at[pl.ds(kk * BLK, BLK)],
                            acca.at[identv.at[kk]], add=True)
        plsc.subcore_barrier()

        rpw = DEG_ROWS // NS
        pltpu.sync_copy(accf.at[pl.ds(s * rpw, rpw)],
                        outf_hbm.at[pl.ds(s * rpw, rpw)])
        pltpu.sync_copy(acca.at[pl.ds(s * rpw, rpw)],
                        outa_hbm.at[pl.ds(s * rpw, rpw)])

    return k(rows_feat, rows_adj, ident, zeros_deg)


KB = 10
NCHUNK = NB // KB
NBLK_TOT = E_PAD // BLK


def _make_agg(vt):
    @jax.jit
    def agg(table, cols, ldst, zeros_half, params):
        @functools.partial(
            pl.kernel,
            out_type=_f32((NP, EMB)),
            mesh=_mesh,
            compiler_params=_cp,
            scratch_types=[
                pltpu.VMEM((KB, BLK), jnp.int32),
                pltpu.VMEM((KB, BLK), jnp.int32),
                pltpu.VMEM((KB, BLK), jnp.int32),
                pltpu.VMEM((KB, BLK), jnp.int32),
                pltpu.VMEM((16,), jnp.int32),
                pltpu.VMEM((BLK, EMB), jnp.float32),
                pltpu.VMEM((BLK, EMB), jnp.float32),
                pltpu.VMEM((BLK, EMB), jnp.float32),
                pltpu.VMEM_SHARED((HPAD, EMB), jnp.float32),
                pltpu.SemaphoreType.DMA,
                pltpu.SemaphoreType.DMA,
                pltpu.SemaphoreType.DMA,
                pltpu.SemaphoreType.DMA,
                pltpu.SemaphoreType.DMA,
            ],
        )
        def k(tab_hbm, cols_hbm, ldst_hbm, z_hbm, par_hbm, out_hbm,
              colv0, colv1, ldstv0, ldstv1, pv, rbuf0, rbuf1, rbuf2, acc,
              si0, si1, sg0, sg1, sg2):
            c = lax.axis_index("c")
            s = lax.axis_index("s")
            colv = (colv0, colv1)
            ldstv = (ldstv0, ldstv1)
            rbuf = (rbuf0, rbuf1, rbuf2)
            semi = (si0, si1)
            semg = (sg0, sg1, sg2)

            pltpu.sync_copy(z_hbm.at[pl.ds(s * STRIPE, STRIPE)],
                            acc.at[pl.ds(s * STRIPE, STRIPE)])
            pltpu.sync_copy(par_hbm, pv)
            pvv = pv[...]
            start = jnp.where(c == 0, pvv[0], pvv[2])
            nch = jnp.where(c == 0, pvv[1], pvv[3])
            cnt = jnp.maximum(0, (nch - s + NS - 1) // NS)

            def idx_args(j, slot):
                blk0 = (start + s + NS * j) * KB
                return ((cols_hbm.at[pl.ds(blk0, KB)],
                         colv[slot], semi[slot]),
                        (ldst_hbm.at[pl.ds(c * NBLK_TOT + blk0, KB)],
                         ldstv[slot], semi[slot]))

            def idx_load(k_, slot):
                a, b = idx_args(k_, slot)
                pltpu.async_copy(*a)
                pltpu.async_copy(*b)

            def idx_wait(k_, slot):
                a, b = idx_args(k_, slot)
                pltpu.make_async_copy(*a).wait()
                pltpu.make_async_copy(*b).wait()

            def g_args(k_, slot, kb, rs):
                return (tab_hbm.at[colv[slot].at[kb]],
                        rbuf[rs], semg[rs])

            @pl.when(cnt > 0)
            def _():
                idx_load(0, 0)

            @pl.when(cnt > 1)
            def _():
                idx_load(1, 1)

            plsc.subcore_barrier()

            @pl.loop(0, (cnt + 1) // 2)
            def _(kp):
                for par in (0, 1):
                    j = kp * 2 + par

                    @pl.when(j < cnt)
                    def _():
                        idx_wait(j, par)
                        pltpu.async_copy(*g_args(j, par, 0, 0))
                        pltpu.async_copy(*g_args(j, par, 1, 1))
                        for kb in range(KB):
                            rs = kb % 3
                            if kb + 2 < KB:
                                pltpu.async_copy(
                                    *g_args(j, par, kb + 2, (kb + 2) % 3))
                            pltpu.make_async_copy(
                                *g_args(j, par, kb, rs)).wait()
                            pltpu.sync_copy(rbuf[rs],
                                            acc.at[ldstv[par].at[kb]],
                                            add=True)

                        @pl.when(j + 2 < cnt)
                        def _():
                            idx_load(j + 2, par)

            plsc.subcore_barrier()
            pltpu.sync_copy(
                acc.at[pl.ds(s * STRIPE, STRIPE)],
                out_hbm.at[pl.ds(c * HPAD + s * STRIPE, STRIPE)])

        return k(table, cols, ldst, zeros_half, params)

    return agg


_agg_feat = _make_agg(FEAT_COLS)
_agg_adj = _make_agg(NP)


@jax.jit
def _batch_gather(table, bidx):
    NW = NC * NS

    @functools.partial(
        pl.kernel,
        out_type=(_f32((BATCH, EMB)), _f32((BATCH, EMB)), _f32((BATCH, EMB))),
        mesh=_mesh,
        compiler_params=_cp,
        scratch_types=[
            pltpu.VMEM((3, BLK), jnp.int32),
            pltpu.VMEM((BLK, EMB), jnp.float32),
            pltpu.SemaphoreType.DMA,
        ],
    )
    def k(tab_hbm, idx_hbm, o0_hbm, o1_hbm, o2_hbm, idxv, rbuf, sem):
        c = lax.axis_index("c")
        s = lax.axis_index("s")
        wid = s * NC + c
        outs = (o0_hbm, o1_hbm, o2_hbm)
        for j in range(3):
            pltpu.sync_copy(idx_hbm.at[pl.ds(wid + 32 * j, 1)],
                            idxv.at[pl.ds(j, 1)])
        for j in range(3):
            pltpu.async_copy(tab_hbm.at[idxv.at[j]], rbuf, sem).wait()
            pltpu.sync_copy(rbuf, outs[j].at[pl.ds(wid * BLK, BLK)])

    return k(table, bidx)


@jax.jit
def _l2_norms(u, p, n):
    def body(u_ref, p_ref, n_ref, o_ref):
        uu = u_ref[...]
        pp = p_ref[...]
        nn = n_ref[...]
        o_ref[...] = jnp.sum(uu * uu + pp * pp + nn * nn, axis=1,
                             keepdims=True)

    out = pl.pallas_call(body, out_shape=_f32((BATCH, 1)))(u, p, n)
    return out[:, 0]


def _pad_half_layout(v):
    return jnp.pad(v.reshape(2, HALF), ((0, 0), (0, HPAD - HALF))).reshape(-1)


def _ldst_both(rows):
    l0 = jnp.where(rows < HALF, rows, HALF)
    r1 = rows - HALF
    l1 = jnp.where((r1 >= 0) & (r1 < HALF), r1, HALF)
    return jnp.stack([l0, l1]).reshape(2, E_PAD // BLK, BLK)


@jax.jit
def kernel(embedding_weight, feat_row, feat_col, adj_row, adj_col,
           users, pos_items, neg_items):
    i32 = jnp.int32
    pad_n = E_PAD - E
    rows_f = jnp.concatenate([feat_row, jnp.full((pad_n,), SENT, i32)])
    rows_a = jnp.concatenate([adj_row, jnp.full((pad_n,), SENT, i32)])
    cols_f = jnp.concatenate(
        [feat_col, jnp.zeros((pad_n,), i32)]).reshape(NBLK_TOT, BLK)
    ldst_f = _ldst_both(rows_f).reshape(2 * NBLK_TOT, BLK)

    adj_col_p = adj_col + 8 * (adj_col >= HALF).astype(i32)
    key = (adj_row >= HALF).astype(i32)
    n0 = jnp.int32(E) - jnp.sum(key, dtype=i32)
    _, rows_s, cols_s = lax.sort((key, adj_row, adj_col_p), num_keys=1,
                                 is_stable=True)
    rows_as = jnp.concatenate([rows_s, jnp.full((pad_n,), SENT, i32)])
    cols_a = jnp.concatenate(
        [cols_s, jnp.zeros((pad_n,), i32)]).reshape(NBLK_TOT, BLK)
    ldst_a = _ldst_both(rows_as).reshape(2 * NBLK_TOT, BLK)

    EPC = KB * BLK
    NCH_TOT = NBLK_TOT // KB
    c0n = (n0 + EPC - 1) // EPC
    c1s = n0 // EPC
    params_adj = (jnp.zeros((16,), i32).at[1].set(c0n)
                  .at[2].set(c1s).at[3].set(NCH_TOT - c1s))
    params_feat = jnp.zeros((16,), i32).at[1].set(NCH_TOT).at[3].set(NCH_TOT)

    ident = jnp.arange(DEG_ROWS, dtype=i32).reshape(DEG_ROWS // BLK, BLK)
    zeros_deg = jnp.zeros((DEG_ROWS, 16), jnp.float32)
    zeros_half = jnp.zeros((HPAD, EMB), jnp.float32)

    degf2, dega2 = _degrees(rows_f, rows_a, ident, zeros_deg)
    deg_f = degf2.reshape(-1)[:N_TOTAL]
    deg_a = dega2.reshape(-1)[:N_TOTAL]

    x_raw = _agg_feat(embedding_weight, cols_f, ldst_f, zeros_half,
                      params_feat)
    f_feat = _pad_half_layout(jnp.where(deg_f > 0, 1.0 / deg_f, 0.0))
    x = x_raw * f_feat[:, None]

    g_a = _pad_half_layout(
        jnp.where(deg_a > 0, 1.0 / jnp.sqrt(deg_a), 0.0))

    rep_sum = x
    h = x
    for _ in range(N_LAYERS):
        s_raw = _agg_adj(h * g_a[:, None], cols_a, ldst_a, zeros_half,
                         params_adj)
        h = s_raw * g_a[:, None]
        rep_sum = rep_sum + h
    final_rep = rep_sum * 0.25

    upos = users
    ppos = HPAD + pos_items
    npos = HPAD + neg_items
    bidx = jnp.concatenate([upos, ppos, npos]).reshape(
        (3 * BATCH) // BLK, BLK)
    users_r, pos_items_r, neg_items_r = _batch_gather(final_rep, bidx)
    l2_norm_sq = _l2_norms(users_r, pos_items_r, neg_items_r)
    return users_r, pos_items_r, neg_items_r, l2_norm_sq
